# baseline (speedup 1.0000x reference)
import functools

import jax
import jax.numpy as jnp
from jax import lax
from jax.experimental import pallas as pl
from jax.experimental.pallas import tpu as pltpu
from jax.experimental.pallas import tpu_sc as plsc

N = 10000
D = 128
E = 320000
NCORE = 2
NSUB = 16
NW = NCORE * NSUB
EPW = E // NW
CH = 128
NFULL = EPW // CH
REM = EPW - NFULL * CH
NPAD = 10240
RPS = NPAD // NSUB
ZR = 40

def _deg_kernel_body(dst_hbm, out_hbm, idx_v, deg_v):
    c = lax.axis_index("c")
    s = lax.axis_index("s")
    wid = s * NCORE + c
    base = wid * EPW
    zeros16 = jnp.zeros((16,), jnp.float32)
    ones16 = jnp.ones((16,), jnp.float32)

    pltpu.sync_copy(dst_hbm.at[pl.ds(base, EPW)], idx_v)

    def zero_body(i, carry):
        deg_v[pl.ds(i * 16, 16)] = zeros16
        return carry

    lax.fori_loop(0, NPAD // 16, zero_body, 0)

    def inner(j, carry):
        idx = idx_v[pl.ds(j * 16, 16)]
        plsc.addupdate_scatter(deg_v, [idx], ones16)
        return carry

    lax.fori_loop(0, EPW // 16, inner, 0)

    pltpu.sync_copy(deg_v, out_hbm.at[wid])


def _agg_kernel_body(g_hbm, src_hbm, dst_hbm, z_hbm, out_hbm,
                     srca_v, dsta_v, rowsa_v, srcb_v, dstb_v, rowsb_v,
                     src_r, dst_r, rows_r,
                     acc_sh, sema, semb, semr):
    c = lax.axis_index("c")
    s = lax.axis_index("s")
    wid = s * NCORE + c
    base = wid * EPW

    pltpu.sync_copy(src_hbm.at[pl.ds(base, CH)], srca_v)
    pltpu.async_copy(g_hbm.at[srca_v], rowsa_v, sema)
    pltpu.sync_copy(dst_hbm.at[pl.ds(base, CH)], dsta_v)

    pltpu.sync_copy(z_hbm.at[pl.ds(s * RPS, RPS)],
                    acc_sh.at[pl.ds(s * RPS, RPS)])
    plsc.subcore_barrier()

    def pair_body(k, carry):
        i1 = 2 * k + 1
        pltpu.sync_copy(src_hbm.at[pl.ds(base + i1 * CH, CH)], srcb_v)
        pltpu.async_copy(g_hbm.at[srcb_v], rowsb_v, semb)
        pltpu.sync_copy(dst_hbm.at[pl.ds(base + i1 * CH, CH)], dstb_v)

        pltpu.make_async_copy(g_hbm.at[srca_v], rowsa_v, sema).wait()
        pltpu.sync_copy(rowsa_v, acc_sh.at[dsta_v], add=True)

        @pl.when(k < NFULL // 2 - 1)
        def _():
            i2 = 2 * k + 2
            pltpu.sync_copy(src_hbm.at[pl.ds(base + i2 * CH, CH)], srca_v)
            pltpu.async_copy(g_hbm.at[srca_v], rowsa_v, sema)
            pltpu.sync_copy(dst_hbm.at[pl.ds(base + i2 * CH, CH)], dsta_v)

        pltpu.make_async_copy(g_hbm.at[srcb_v], rowsb_v, semb).wait()
        pltpu.sync_copy(rowsb_v, acc_sh.at[dstb_v], add=True)
        return carry

    lax.fori_loop(0, NFULL // 2, pair_body, 0)

    pltpu.sync_copy(src_hbm.at[pl.ds(base + NFULL * CH, REM)], src_r)
    pltpu.sync_copy(dst_hbm.at[pl.ds(base + NFULL * CH, REM)], dst_r)
    pltpu.async_copy(g_hbm.at[src_r], rows_r, semr).wait()
    pltpu.sync_copy(rows_r, acc_sh.at[dst_r], add=True)

    plsc.subcore_barrier()
    pltpu.sync_copy(acc_sh.at[pl.ds(s * RPS, RPS)],
                    out_hbm.at[c, pl.ds(s * RPS, RPS)])


def _tc0_body(x_ref, w1_ref, h_ref):
    h_ref[...] = jnp.dot(x_ref[...], w1_ref[...],
                         preferred_element_type=jnp.float32)


def _tc1_body(h_ref, parts_ref, dis_ref, g_ref):
    deg = jnp.sum(parts_ref[...], axis=0)[:N] + 1.0
    dis = lax.rsqrt(deg)[:, None]
    dis_ref[...] = dis
    g_ref[...] = h_ref[...] * dis


def _tc2_body(p_ref, g_ref, dis_ref, b1_ref, w2_ref, g2_ref):
    agg = p_ref[0, :N] + p_ref[1, :N] + g_ref[...]
    t = jnp.maximum(agg * dis_ref[...] + b1_ref[...], 0.0)
    h2 = jnp.dot(t, w2_ref[...], preferred_element_type=jnp.float32)
    g2_ref[...] = h2 * dis_ref[...]


def _tc3_body(p_ref, g_ref, dis_ref, b2_ref, wlin_ref, blin_ref, out_ref):
    o = (p_ref[0, :N] + p_ref[1, :N] + g_ref[...]) * dis_ref[...] + b2_ref[...]
    logits = jnp.dot(o, wlin_ref[...], preferred_element_type=jnp.float32)
    logits = logits + blin_ref[...]
    m = jnp.max(logits, axis=1, keepdims=True)
    lse = jnp.log(jnp.sum(jnp.exp(logits - m), axis=1, keepdims=True))
    out_ref[...] = logits - m - lse


_tc0 = pl.pallas_call(
    _tc0_body,
    out_shape=jax.ShapeDtypeStruct((N, D), jnp.float32),
)
_tc1 = pl.pallas_call(
    _tc1_body,
    out_shape=(jax.ShapeDtypeStruct((N, 1), jnp.float32),
               jax.ShapeDtypeStruct((N, D), jnp.float32)),
)
_tc2 = pl.pallas_call(
    _tc2_body,
    out_shape=jax.ShapeDtypeStruct((N, D), jnp.float32),
)
_tc3 = pl.pallas_call(
    _tc3_body,
    out_shape=jax.ShapeDtypeStruct((N, 5), jnp.float32),
)


@functools.cache
def _build_sc_kernels():
    mesh = plsc.VectorSubcoreMesh(core_axis_name="c", subcore_axis_name="s",
                                  num_cores=NCORE, num_subcores=NSUB)
    params = pltpu.CompilerParams(needs_layout_passes=False)
    deg_kernel = pl.kernel(
        _deg_kernel_body,
        out_type=jax.ShapeDtypeStruct((NW, NPAD), jnp.float32),
        mesh=mesh,
        compiler_params=params,
        scratch_types=[
            pltpu.VMEM((EPW,), jnp.int32),
            pltpu.VMEM((NPAD,), jnp.float32),
        ],
    )
    agg_kernel = pl.kernel(
        _agg_kernel_body,
        out_type=jax.ShapeDtypeStruct((NCORE, NPAD, D), jnp.float32),
        mesh=mesh,
        compiler_params=params,
        scratch_types=[
            pltpu.VMEM((CH,), jnp.int32),
            pltpu.VMEM((CH,), jnp.int32),
            pltpu.VMEM((CH, D), jnp.float32),
            pltpu.VMEM((CH,), jnp.int32),
            pltpu.VMEM((CH,), jnp.int32),
            pltpu.VMEM((CH, D), jnp.float32),
            pltpu.VMEM((REM,), jnp.int32),
            pltpu.VMEM((REM,), jnp.int32),
            pltpu.VMEM((REM, D), jnp.float32),
            pltpu.VMEM_SHARED((NPAD, D), jnp.float32),
            pltpu.SemaphoreType.DMA,
            pltpu.SemaphoreType.DMA,
            pltpu.SemaphoreType.DMA,
        ],
    )
    return deg_kernel, agg_kernel


def kernel(x, edge_index, W1, b1, W2, b2, Wlin, blin):
    _deg_kernel, _agg_kernel = _build_sc_kernels()
    src = edge_index[0].astype(jnp.int32)
    dst = edge_index[1].astype(jnp.int32)
    zeros = jnp.zeros((NPAD, D), jnp.float32)
    deg_parts = _deg_kernel(dst)
    h1 = _tc0(x, W1)
    dis, g1 = _tc1(h1, deg_parts)
    p1 = _agg_kernel(g1, src, dst, zeros)
    g2 = _tc2(p1, g1, dis, b1, W2)
    p2 = _agg_kernel(g2, src, dst, zeros)
    return _tc3(p2, g2, dis, b2, Wlin, blin)


# speedup vs baseline: 1.0039x; 1.0039x over previous
import functools

import jax
import jax.numpy as jnp
from jax import lax
from jax.experimental import pallas as pl
from jax.experimental.pallas import tpu as pltpu
from jax.experimental.pallas import tpu_sc as plsc

N = 10000
D = 128
E = 320000
NCORE = 2
NSUB = 16
NW = NCORE * NSUB
EPW = E // NW
CH = 128
NFULL = EPW // CH
REM = EPW - NFULL * CH
NPAD = 10240
RPS = NPAD // NSUB
ZR = 40

def _deg_kernel_body(dst_hbm, out_hbm, idx_v, deg_v):
    c = lax.axis_index("c")
    s = lax.axis_index("s")
    wid = s * NCORE + c
    base = wid * EPW
    zeros16 = jnp.zeros((16,), jnp.float32)
    ones16 = jnp.ones((16,), jnp.float32)

    pltpu.sync_copy(dst_hbm.at[pl.ds(base, EPW)], idx_v)

    def zero_body(i, carry):
        deg_v[pl.ds(i * 16, 16)] = zeros16
        return carry

    lax.fori_loop(0, NPAD // 16, zero_body, 0)

    def inner(j, carry):
        idx = idx_v[pl.ds(j * 16, 16)]
        plsc.addupdate_scatter(deg_v, [idx], ones16)
        return carry

    lax.fori_loop(0, EPW // 16, inner, 0)

    pltpu.sync_copy(deg_v, out_hbm.at[wid])


def _agg_kernel_body(g_hbm, src_hbm, dst_hbm, z_hbm, out_hbm,
                     srca_v, dsta_v, rowsa_v, srcb_v, dstb_v, rowsb_v,
                     src_r, dst_r, rows_r,
                     acc_sh, sema, semb, semr):
    c = lax.axis_index("c")
    s = lax.axis_index("s")
    wid = s * NCORE + c
    base = wid * EPW

    pltpu.sync_copy(src_hbm.at[pl.ds(base, CH)], srca_v)
    pltpu.async_copy(g_hbm.at[srca_v], rowsa_v, sema)
    pltpu.sync_copy(dst_hbm.at[pl.ds(base, CH)], dsta_v)

    pltpu.sync_copy(z_hbm.at[pl.ds(s * RPS, RPS)],
                    acc_sh.at[pl.ds(s * RPS, RPS)])
    plsc.subcore_barrier()

    def pair_body(k, carry):
        i1 = 2 * k + 1
        pltpu.sync_copy(src_hbm.at[pl.ds(base + i1 * CH, CH)], srcb_v)
        pltpu.async_copy(g_hbm.at[srcb_v], rowsb_v, semb)
        pltpu.sync_copy(dst_hbm.at[pl.ds(base + i1 * CH, CH)], dstb_v)

        pltpu.make_async_copy(g_hbm.at[srca_v], rowsa_v, sema).wait()
        pltpu.sync_copy(rowsa_v, acc_sh.at[dsta_v], add=True)

        @pl.when(k < NFULL // 2 - 1)
        def _():
            i2 = 2 * k + 2
            pltpu.sync_copy(src_hbm.at[pl.ds(base + i2 * CH, CH)], srca_v)
            pltpu.async_copy(g_hbm.at[srca_v], rowsa_v, sema)
            pltpu.sync_copy(dst_hbm.at[pl.ds(base + i2 * CH, CH)], dsta_v)

        pltpu.make_async_copy(g_hbm.at[srcb_v], rowsb_v, semb).wait()
        pltpu.sync_copy(rowsb_v, acc_sh.at[dstb_v], add=True)
        return carry

    lax.fori_loop(0, NFULL // 2, pair_body, 0)

    pltpu.sync_copy(src_hbm.at[pl.ds(base + NFULL * CH, REM)], src_r)
    pltpu.sync_copy(dst_hbm.at[pl.ds(base + NFULL * CH, REM)], dst_r)
    pltpu.async_copy(g_hbm.at[src_r], rows_r, semr).wait()
    pltpu.sync_copy(rows_r, acc_sh.at[dst_r], add=True)

    plsc.subcore_barrier()
    pltpu.sync_copy(acc_sh.at[pl.ds(s * RPS, RPS)],
                    out_hbm.at[c, pl.ds(s * RPS, RPS)])


def _tc1_body(x_ref, w1_ref, parts_ref, dis_ref, g_ref):
    deg = jnp.sum(parts_ref[...], axis=0)[:N] + 1.0
    dis = lax.rsqrt(deg)[:, None]
    h = jnp.dot(x_ref[...], w1_ref[...], preferred_element_type=jnp.float32)
    dis_ref[...] = dis
    g_ref[...] = h * dis


def _tc2_body(p_ref, g_ref, dis_ref, b1_ref, w2_ref, g2_ref):
    agg = p_ref[0, :N] + p_ref[1, :N] + g_ref[...]
    t = jnp.maximum(agg * dis_ref[...] + b1_ref[...], 0.0)
    h2 = jnp.dot(t, w2_ref[...], preferred_element_type=jnp.float32)
    g2_ref[...] = h2 * dis_ref[...]


def _tc3_body(p_ref, g_ref, dis_ref, b2_ref, wlin_ref, blin_ref, out_ref):
    o = (p_ref[0, :N] + p_ref[1, :N] + g_ref[...]) * dis_ref[...] + b2_ref[...]
    logits = jnp.dot(o, wlin_ref[...], preferred_element_type=jnp.float32)
    logits = logits + blin_ref[...]
    m = jnp.max(logits, axis=1, keepdims=True)
    lse = jnp.log(jnp.sum(jnp.exp(logits - m), axis=1, keepdims=True))
    out_ref[...] = logits - m - lse


_tc1 = pl.pallas_call(
    _tc1_body,
    out_shape=(jax.ShapeDtypeStruct((N, 1), jnp.float32),
               jax.ShapeDtypeStruct((N, D), jnp.float32)),
)
_tc2 = pl.pallas_call(
    _tc2_body,
    out_shape=jax.ShapeDtypeStruct((N, D), jnp.float32),
)
_tc3 = pl.pallas_call(
    _tc3_body,
    out_shape=jax.ShapeDtypeStruct((N, 5), jnp.float32),
)


@functools.cache
def _build_sc_kernels():
    mesh = plsc.VectorSubcoreMesh(core_axis_name="c", subcore_axis_name="s",
                                  num_cores=NCORE, num_subcores=NSUB)
    params = pltpu.CompilerParams(needs_layout_passes=False)
    deg_kernel = pl.kernel(
        _deg_kernel_body,
        out_type=jax.ShapeDtypeStruct((NW, NPAD), jnp.float32),
        mesh=mesh,
        compiler_params=params,
        scratch_types=[
            pltpu.VMEM((EPW,), jnp.int32),
            pltpu.VMEM((NPAD,), jnp.float32),
        ],
    )
    agg_kernel = pl.kernel(
        _agg_kernel_body,
        out_type=jax.ShapeDtypeStruct((NCORE, NPAD, D), jnp.float32),
        mesh=mesh,
        compiler_params=params,
        scratch_types=[
            pltpu.VMEM((CH,), jnp.int32),
            pltpu.VMEM((CH,), jnp.int32),
            pltpu.VMEM((CH, D), jnp.float32),
            pltpu.VMEM((CH,), jnp.int32),
            pltpu.VMEM((CH,), jnp.int32),
            pltpu.VMEM((CH, D), jnp.float32),
            pltpu.VMEM((REM,), jnp.int32),
            pltpu.VMEM((REM,), jnp.int32),
            pltpu.VMEM((REM, D), jnp.float32),
            pltpu.VMEM_SHARED((NPAD, D), jnp.float32),
            pltpu.SemaphoreType.DMA,
            pltpu.SemaphoreType.DMA,
            pltpu.SemaphoreType.DMA,
        ],
    )
    return deg_kernel, agg_kernel


def kernel(x, edge_index, W1, b1, W2, b2, Wlin, blin):
    _deg_kernel, _agg_kernel = _build_sc_kernels()
    src = edge_index[0].astype(jnp.int32)
    dst = edge_index[1].astype(jnp.int32)
    zeros = jnp.zeros((NPAD, D), jnp.float32)
    deg_parts = _deg_kernel(dst)
    dis, g1 = _tc1(x, W1, deg_parts)
    p1 = _agg_kernel(g1, src, dst, zeros)
    g2 = _tc2(p1, g1, dis, b1, W2)
    p2 = _agg_kernel(g2, src, dst, zeros)
    return _tc3(p2, g2, dis, b2, Wlin, blin)
